# Initial kernel scaffold; baseline (speedup 1.0000x reference)
import jax
import jax.numpy as jnp
from jax.experimental import pallas as pl


def kernel(x, table):
    raise NotImplementedError("write your pallas kernel here")


# speedup vs baseline: 1.0161x; 1.0161x over previous
import functools

import jax
import jax.numpy as jnp
from jax import lax
from jax.experimental import pallas as pl
from jax.experimental.pallas import tpu as pltpu
from jax.experimental.pallas import tpu_sc as plsc

DIM = 32
SCALE = float(DIM ** 0.5)

NUM_CORES = 2
NUM_SUBCORES = 16
NW = NUM_CORES * NUM_SUBCORES

IDX_W = 128
CHUNK_IDX_ROWS = 8
CHUNK = CHUNK_IDX_ROWS * IDX_W


def _make_kernel(B):
    assert B % (NW * CHUNK) == 0
    b_per_w = B // NW
    n_chunks = b_per_w // CHUNK
    idx_rows_per_w = b_per_w // IDX_W

    mesh = plsc.VectorSubcoreMesh(core_axis_name="c", subcore_axis_name="s")

    @functools.partial(
        pl.kernel,
        mesh=mesh,
        out_type=jax.ShapeDtypeStruct((B, DIM), jnp.float32),
        scratch_types=[
            pltpu.VMEM((CHUNK_IDX_ROWS, IDX_W), jnp.int32),
            pltpu.VMEM((CHUNK, DIM), jnp.float32),
            pltpu.SemaphoreType.DMA,
        ],
        compiler_params=pltpu.CompilerParams(use_tc_tiling_on_sc=False),
    )
    def emb(table_hbm, idx_hbm, out_hbm, idx_v, rows_v, sem):
        wid = lax.axis_index("s") * NUM_CORES + lax.axis_index("c")
        idx_row_base = wid * idx_rows_per_w
        out_base = wid * b_per_w

        def chunk_body(ci, carry):
            pltpu.sync_copy(
                idx_hbm.at[pl.ds(idx_row_base + ci * CHUNK_IDX_ROWS,
                                 CHUNK_IDX_ROWS)],
                idx_v,
            )
            copies = []
            for j in range(CHUNK_IDX_ROWS):
                copies.append(
                    pltpu.async_copy(
                        table_hbm.at[idx_v.at[j]],
                        rows_v.at[pl.ds(j * IDX_W, IDX_W)],
                        sem,
                    )
                )
            for c in copies:
                c.wait()

            def scale_body(i, c):
                rows_v[i, pl.ds(0, 16)] = rows_v[i, pl.ds(0, 16)] * SCALE
                rows_v[i, pl.ds(16, 16)] = rows_v[i, pl.ds(16, 16)] * SCALE
                return c

            lax.fori_loop(0, CHUNK, scale_body, 0, unroll=4)

            pltpu.sync_copy(
                rows_v,
                out_hbm.at[pl.ds(out_base + ci * CHUNK, CHUNK)],
            )
            return carry

        lax.fori_loop(0, n_chunks, chunk_body, 0)

    return emb


def kernel(x, table):
    S0, S1 = x.shape
    B = S0 * S1
    idx = x.reshape(B // IDX_W, IDX_W).astype(jnp.int32)
    out = _make_kernel(B)(table, idx)
    return out.reshape(S0, S1, DIM)


# speedup vs baseline: 1.5549x; 1.5304x over previous
import functools

import jax
import jax.numpy as jnp
from jax import lax
from jax.experimental import pallas as pl
from jax.experimental.pallas import tpu as pltpu
from jax.experimental.pallas import tpu_sc as plsc

DIM = 32
SCALE = float(DIM ** 0.5)

NUM_CORES = 2
NUM_SUBCORES = 16
NW = NUM_CORES * NUM_SUBCORES

CHUNK_ROWS = 16


def _make_kernel(S0, S1):
    assert S0 % (NW * CHUNK_ROWS) == 0
    rows_per_w = S0 // NW
    n_chunks = rows_per_w // CHUNK_ROWS

    mesh = plsc.VectorSubcoreMesh(core_axis_name="c", subcore_axis_name="s")

    @functools.partial(
        pl.kernel,
        mesh=mesh,
        out_type=jax.ShapeDtypeStruct((S0, S1, DIM), jnp.float32),
        scratch_types=[
            pltpu.VMEM((CHUNK_ROWS, S1), jnp.int32),
            pltpu.VMEM((CHUNK_ROWS, S1, DIM), jnp.float32),
            pltpu.SemaphoreType.DMA,
        ],
        compiler_params=pltpu.CompilerParams(use_tc_tiling_on_sc=False),
    )
    def emb(x_hbm, table_hbm, out_hbm, idx_v, rows_v, sem):
        wid = lax.axis_index("s") * NUM_CORES + lax.axis_index("c")
        row_base = wid * rows_per_w

        def chunk_body(ci, carry):
            row0 = row_base + ci * CHUNK_ROWS
            pltpu.sync_copy(x_hbm.at[pl.ds(row0, CHUNK_ROWS)], idx_v)
            copies = []
            for r in range(CHUNK_ROWS):
                copies.append(
                    pltpu.async_copy(
                        table_hbm.at[idx_v.at[r]], rows_v.at[r], sem))
            for c in copies:
                c.wait()

            def scale_body(r, c):
                def col_body(j, c2):
                    rows_v[r, j, pl.ds(0, 16)] = (
                        rows_v[r, j, pl.ds(0, 16)] * SCALE)
                    rows_v[r, j, pl.ds(16, 16)] = (
                        rows_v[r, j, pl.ds(16, 16)] * SCALE)
                    return c2
                return lax.fori_loop(0, S1, col_body, c, unroll=2)

            lax.fori_loop(0, CHUNK_ROWS, scale_body, 0)

            pltpu.sync_copy(rows_v, out_hbm.at[pl.ds(row0, CHUNK_ROWS)])
            return carry

        lax.fori_loop(0, n_chunks, chunk_body, 0)

    return emb


def kernel(x, table):
    S0, S1 = x.shape
    return _make_kernel(S0, S1)(x.astype(jnp.int32), table)
